# baseline (speedup 1.0000x reference)
import functools

import jax
import jax.numpy as jnp
from jax import lax
from jax.experimental import pallas as pl
from jax.experimental.pallas import tpu as pltpu
from jax.experimental.pallas import tpu_sc as plsc

DIM = 128
NC, NS = 2, 16
NW = NC * NS
CHUNK = 256
REGION = CHUNK + 32


def _fold_kernel(emb_ref, w_ref, b_ref, out_ref):
    out_ref[...] = lax.dot_general(
        emb_ref[...], w_ref[...],
        dimension_numbers=(((1,), (1,)), ((), ())),
        preferred_element_type=jnp.float32,
    ) + b_ref[...]


def _fold_table(emb_pad, W, b):
    rows = emb_pad.shape[0]
    return pl.pallas_call(
        _fold_kernel,
        out_shape=jax.ShapeDtypeStruct((rows, DIM), jnp.float32),
    )(emb_pad, W, b.reshape(1, DIM))


@functools.lru_cache(maxsize=None)
def _make_gather(n_total, rows, vocab):
    assert n_total % (NW * CHUNK) == 0
    per_w = n_total // NW
    n_chunks = per_w // CHUNK
    assert n_chunks % 2 == 0
    buf_f = (CHUNK + 1) * DIM
    chunk_f = CHUNK * DIM
    mesh = plsc.VectorSubcoreMesh(
        core_axis_name="c", subcore_axis_name="s",
        num_cores=NC, num_subcores=NS)

    @functools.partial(
        pl.kernel, mesh=mesh,
        out_type=jax.ShapeDtypeStruct((n_total * DIM,), jnp.float32),
        scratch_types=[
            pltpu.VMEM((rows * DIM,), jnp.float32),
            pltpu.VMEM((per_w,), jnp.int32),
            pltpu.VMEM((vocab * REGION,), jnp.int32),
            pltpu.VMEM((16,), jnp.int32),
            pltpu.VMEM((buf_f,), jnp.float32),
            pltpu.VMEM((buf_f,), jnp.float32),
            pltpu.SemaphoreType.DMA,
            pltpu.SemaphoreType.DMA,
        ],
        compiler_params=pltpu.CompilerParams(needs_layout_passes=False),
    )
    def gather(table_hbm, idx_hbm, out_hbm, table_v, idx_v, pos_v, cnt_v,
               buf_a, buf_b, sa, sb):
        wid = lax.axis_index("s") * NC + lax.axis_index("c")
        base = wid * per_w
        pltpu.sync_copy(table_hbm, table_v)
        pltpu.sync_copy(idx_hbm.at[pl.ds(base, per_w)], idx_v)
        iota = lax.iota(jnp.int32, 16)
        dummy = jnp.full((16,), CHUNK, jnp.int32)
        ones = jnp.full((16,), 1, jnp.int32)
        zeros = jnp.zeros((16,), jnp.int32)

        def build(buf, t):
            cnt_v[pl.ds(0, 16)] = zeros

            @pl.loop(0, CHUNK // 16)
            def _(g):
                idxv = idx_v[pl.ds(t * CHUNK + g * 16, 16)]
                posv = iota + g * 16
                rank, _last = plsc.scan_count(idxv)
                basev = plsc.load_gather(cnt_v, [idxv])
                dst = idxv * REGION + basev + (rank - 1)
                plsc.store_scatter(pos_v, [dst], posv)
                plsc.addupdate_scatter(cnt_v, [idxv], ones)

            cntv = cnt_v[pl.ds(0, 16)]
            for v in range(vocab):
                pos_v[pl.ds(v * REGION + cntv[v], 16)] = dummy
                pos_v[pl.ds(v * REGION + cntv[v] + 16, 16)] = dummy

            for v in range(vocab):
                rowregs = [table_v[pl.ds(v * DIM + c * 16, 16)]
                           for c in range(DIM // 16)]
                for k in range(2):
                    pw = pos_v[pl.ds(v * REGION + k * 8, 16)]
                    for j in range(8):
                        dst = pw[j] * DIM
                        for c in range(DIM // 16):
                            buf[pl.ds(dst + c * 16, 16)] = rowregs[c]
                n8 = (cntv[v] + 7) >> 3

                @plsc.parallel_loop(2, jnp.maximum(n8, 2))
                def _(k, v=v, rowregs=rowregs):
                    pw = pos_v[pl.ds(v * REGION + k * 8, 16)]
                    for j in range(8):
                        dst = pw[j] * DIM
                        for c in range(DIM // 16):
                            buf[pl.ds(dst + c * 16, 16)] = rowregs[c]

        def store_start(buf, t, sem):
            off = (base + t * CHUNK) * DIM
            pltpu.async_copy(
                buf.at[pl.ds(0, chunk_f)], out_hbm.at[pl.ds(off, chunk_f)],
                sem)

        def drain(buf, sem):
            pltpu.make_async_copy(
                buf.at[pl.ds(0, chunk_f)],
                out_hbm.at[pl.ds(base * DIM, chunk_f)], sem).wait()

        @pl.loop(0, n_chunks // 2)
        def _(i):
            t0 = 2 * i

            @pl.when(i > 0)
            def _():
                drain(buf_a, sa)

            build(buf_a, t0)
            store_start(buf_a, t0, sa)

            @pl.when(i > 0)
            def _():
                drain(buf_b, sb)

            build(buf_b, t0 + 1)
            store_start(buf_b, t0 + 1, sb)

        drain(buf_a, sa)
        drain(buf_b, sb)

    return gather


def kernel(x, embed_table, W, b):
    B, C, H, W_ = x.shape
    L = C * H * W_
    idx = x.reshape(-1).astype(jnp.int32)
    vocab = embed_table.shape[0]
    rows = max(8, -(-vocab // 8) * 8)
    emb_pad = jnp.zeros((rows, DIM), embed_table.dtype).at[:vocab].set(embed_table)
    proj = _fold_table(emb_pad, W, b)
    out = _make_gather(idx.shape[0], rows, vocab)(proj.reshape(-1), idx)
    return out.reshape(B, L, DIM)


# speedup vs baseline: 1.9858x; 1.9858x over previous
import functools

import jax
import jax.numpy as jnp
from jax import lax
from jax.experimental import pallas as pl
from jax.experimental.pallas import tpu as pltpu
from jax.experimental.pallas import tpu_sc as plsc

DIM = 128
NC, NS = 2, 16
NW = NC * NS
CHUNK = 256


def _fold_kernel(emb_ref, w_ref, b_ref, out_ref):
    out_ref[...] = lax.dot_general(
        emb_ref[...], w_ref[...],
        dimension_numbers=(((1,), (1,)), ((), ())),
        preferred_element_type=jnp.float32,
    ) + b_ref[...]


def _fold_table(emb_pad, W, b):
    rows = emb_pad.shape[0]
    return pl.pallas_call(
        _fold_kernel,
        out_shape=jax.ShapeDtypeStruct((rows, DIM), jnp.float32),
    )(emb_pad, W, b.reshape(1, DIM))


@functools.lru_cache(maxsize=None)
def _make_gather(n_total, rows):
    assert n_total % (NW * CHUNK) == 0
    per_w = n_total // NW
    n_chunks = per_w // CHUNK
    assert n_chunks % 2 == 0
    mesh = plsc.VectorSubcoreMesh(
        core_axis_name="c", subcore_axis_name="s",
        num_cores=NC, num_subcores=NS)

    @functools.partial(
        pl.kernel, mesh=mesh,
        out_type=jax.ShapeDtypeStruct((n_total * DIM,), jnp.float32),
        scratch_types=[
            pltpu.VMEM((rows * DIM,), jnp.float32),
            pltpu.VMEM((per_w,), jnp.int32),
            pltpu.VMEM((CHUNK * DIM,), jnp.float32),
            pltpu.VMEM((CHUNK * DIM,), jnp.float32),
            pltpu.SemaphoreType.DMA,
            pltpu.SemaphoreType.DMA,
        ],
        compiler_params=pltpu.CompilerParams(needs_layout_passes=False),
    )
    def gather(table_hbm, idx_hbm, out_hbm, table_v, idx_v, buf_a, buf_b, sa, sb):
        wid = lax.axis_index("s") * NC + lax.axis_index("c")
        base = wid * per_w
        pltpu.sync_copy(table_hbm, table_v)
        pltpu.sync_copy(idx_hbm.at[pl.ds(base, per_w)], idx_v)
        iota = lax.iota(jnp.int32, 16)

        def build(buf, t):
            @plsc.parallel_loop(0, CHUNK // 16, unroll=2)
            def _(g):
                rowbase = g * 16
                idxv = idx_v[pl.ds(t * CHUNK + rowbase, 16)]
                for j in range(16):
                    r = idxv[j]
                    src = r * DIM
                    dst = (rowbase + j) * DIM
                    for c in range(DIM // 16):
                        buf[pl.ds(dst + c * 16, 16)] = (
                            table_v[pl.ds(src + c * 16, 16)])

        def store_start(buf, t, sem):
            off = (base + t * CHUNK) * DIM
            pltpu.async_copy(buf, out_hbm.at[pl.ds(off, CHUNK * DIM)], sem)

        def drain(buf, sem):
            pltpu.make_async_copy(
                buf, out_hbm.at[pl.ds(base * DIM, CHUNK * DIM)], sem).wait()

        @pl.loop(0, n_chunks // 2)
        def _(i):
            t0 = 2 * i

            @pl.when(i > 0)
            def _():
                drain(buf_a, sa)

            build(buf_a, t0)
            store_start(buf_a, t0, sa)

            @pl.when(i > 0)
            def _():
                drain(buf_b, sb)

            build(buf_b, t0 + 1)
            store_start(buf_b, t0 + 1, sb)

        drain(buf_a, sa)
        drain(buf_b, sb)

    return gather


def kernel(x, embed_table, W, b):
    B, C, H, W_ = x.shape
    L = C * H * W_
    idx = x.reshape(-1).astype(jnp.int32)
    vocab = embed_table.shape[0]
    rows = max(8, -(-vocab // 8) * 8)
    emb_pad = jnp.zeros((rows, DIM), embed_table.dtype).at[:vocab].set(embed_table)
    proj = _fold_table(emb_pad, W, b)
    out = _make_gather(idx.shape[0], rows)(proj.reshape(-1), idx)
    return out.reshape(B, L, DIM)
